# Initial kernel scaffold; baseline (speedup 1.0000x reference)
import jax
import jax.numpy as jnp
from jax.experimental import pallas as pl


def kernel(s, T):
    raise NotImplementedError("write your pallas kernel here")


# speedup vs baseline: 31.3239x; 31.3239x over previous
import jax
import jax.numpy as jnp
from jax import lax
from jax.experimental import pallas as pl
from jax.experimental.pallas import tpu as pltpu
from jax.experimental.pallas import tpu_sc as plsc

N = 3_200_000
NC = 2
NS = 16
L = 16
NW = NC * NS
P = N // NW
C = 20_000
NCHUNK = P // C
NV = C // L


def _wid():
    return lax.axis_index("s") * NC + lax.axis_index("c")


def _lane_allreduce(v):
    lanes = lax.iota(jnp.int32, L)
    dnums = lax.GatherDimensionNumbers(
        offset_dims=(), collapsed_slice_dims=(0,), start_index_map=(0,))
    for d in (1, 2, 4, 8):
        g = lax.gather(v, (lanes ^ d)[:, None], dnums, slice_sizes=(1,),
                       mode=lax.GatherScatterMode.PROMISE_IN_BOUNDS)
        v = v + g
    return v


def _sum_body(s_hbm, t_hbm, part_hbm, s_buf, t_buf, pvec_buf):
    base = _wid() * P

    def chunk(ci, carry):
        acc_all, acc_1 = carry
        off = base + ci * C
        pltpu.sync_copy(s_hbm.at[pl.ds(off, C)], s_buf)
        pltpu.sync_copy(t_hbm.at[pl.ds(off, C)], t_buf)

        def inner(i, carry2):
            a_all, a_1 = carry2
            v = jnp.exp(s_buf[pl.ds(i * L, L)])
            t = t_buf[pl.ds(i * L, L)]
            return a_all + v, a_1 + jnp.where(t == 1, v, 0.0)

        return lax.fori_loop(0, NV, inner, (acc_all, acc_1))

    z = jnp.zeros((L,), jnp.float32)
    acc_all, acc_1 = lax.fori_loop(0, NCHUNK, chunk, (z, z))
    pvec_buf[pl.ds(0, L)] = acc_all
    pvec_buf[pl.ds(L, L)] = acc_1
    pltpu.sync_copy(pvec_buf, part_hbm.at[pl.ds(_wid() * 2 * L, 2 * L)])


def _norm_body(s_hbm, t_hbm, part_hbm, out_hbm, s_buf, t_buf, o_buf, p_buf):
    pltpu.sync_copy(part_hbm, p_buf)

    def red(i, carry):
        a_all, a_1 = carry
        return (a_all + p_buf[pl.ds(i * 2 * L, L)],
                a_1 + p_buf[pl.ds(i * 2 * L + L, L)])

    z = jnp.zeros((L,), jnp.float32)
    acc_all, acc_1 = lax.fori_loop(0, NW, red, (z, z))
    r1 = _lane_allreduce(acc_1)
    r0 = _lane_allreduce(acc_all) - r1

    base = _wid() * P

    def chunk(ci, _):
        off = base + ci * C
        pltpu.sync_copy(s_hbm.at[pl.ds(off, C)], s_buf)
        pltpu.sync_copy(t_hbm.at[pl.ds(off, C)], t_buf)

        def inner(i, _2):
            v = jnp.exp(s_buf[pl.ds(i * L, L)])
            t = t_buf[pl.ds(i * L, L)]
            o_buf[pl.ds(i * L, L)] = v / jnp.where(t == 1, r1, r0)
            return 0

        lax.fori_loop(0, NV, inner, 0)
        pltpu.sync_copy(o_buf, out_hbm.at[pl.ds(off, C)])
        return 0

    lax.fori_loop(0, NCHUNK, chunk, 0)


def kernel(s, T):
    mesh = plsc.VectorSubcoreMesh(core_axis_name="c", subcore_axis_name="s")
    part = pl.kernel(
        _sum_body,
        mesh=mesh,
        out_type=jax.ShapeDtypeStruct((NW * 2 * L,), jnp.float32),
        scratch_types=[
            pltpu.VMEM((C,), jnp.float32),
            pltpu.VMEM((C,), jnp.int32),
            pltpu.VMEM((2 * L,), jnp.float32),
        ],
    )(s, T)
    out = pl.kernel(
        _norm_body,
        mesh=mesh,
        out_type=jax.ShapeDtypeStruct((N,), jnp.float32),
        scratch_types=[
            pltpu.VMEM((C,), jnp.float32),
            pltpu.VMEM((C,), jnp.int32),
            pltpu.VMEM((C,), jnp.float32),
            pltpu.VMEM((NW * 2 * L,), jnp.float32),
        ],
    )(s, T, part)
    return out


# speedup vs baseline: 60.2490x; 1.9234x over previous
import jax
import jax.numpy as jnp
from jax import lax
from jax.experimental import pallas as pl
from jax.experimental.pallas import tpu as pltpu
from jax.experimental.pallas import tpu_sc as plsc

N = 3_200_000
NC = 2
NS = 16
L = 16
NW = NC * NS
P = N // NW
C = 20_000
NCHUNK = P // C
NV = C // L
U = 5


def _wid():
    return lax.axis_index("s") * NC + lax.axis_index("c")


def _lane_allreduce(v):
    lanes = lax.iota(jnp.int32, L)
    dnums = lax.GatherDimensionNumbers(
        offset_dims=(), collapsed_slice_dims=(0,), start_index_map=(0,))
    for d in (1, 2, 4, 8):
        g = lax.gather(v, (lanes ^ d)[:, None], dnums, slice_sizes=(1,),
                       mode=lax.GatherScatterMode.PROMISE_IN_BOUNDS)
        v = v + g
    return v


def _sum_body(s_hbm, t_hbm, part_hbm, s_buf0, s_buf1, t_buf0, t_buf1,
              pvec_buf, sem0, sem1):
    base = _wid() * P
    sems = (sem0, sem1)
    s_bufs = (s_buf0, s_buf1)
    t_bufs = (t_buf0, t_buf1)

    def start(ci):
        slot = ci % 2
        off = base + ci * C
        pltpu.async_copy(s_hbm.at[pl.ds(off, C)], s_bufs[slot], sems[slot])
        return pltpu.async_copy(
            t_hbm.at[pl.ds(off, C)], t_bufs[slot], sems[slot])

    h = start(0)
    z = jnp.zeros((L,), jnp.float32)
    accs = tuple((z, z) for _ in range(U))
    for ci in range(NCHUNK):
        h.wait()
        h.wait()
        if ci + 1 < NCHUNK:
            h = start(ci + 1)
        slot = ci % 2

        sb, tb = s_bufs[slot], t_bufs[slot]

        @plsc.parallel_loop(0, NV, step=U, carry=accs)
        def accs(i, carry):
            out = []
            for j in range(U):
                a_all, a_1 = carry[j]
                v = jnp.exp(sb[pl.ds((i + j) * L, L)])
                t = tb[pl.ds((i + j) * L, L)]
                out.append((a_all + v, a_1 + jnp.where(t == 1, v, 0.0)))
            return tuple(out)

    acc_all = accs[0][0]
    acc_1 = accs[0][1]
    for j in range(1, U):
        acc_all = acc_all + accs[j][0]
        acc_1 = acc_1 + accs[j][1]
    pvec_buf[pl.ds(0, L)] = acc_all
    pvec_buf[pl.ds(L, L)] = acc_1
    pltpu.sync_copy(pvec_buf, part_hbm.at[pl.ds(_wid() * 2 * L, 2 * L)])


def _norm_body(s_hbm, t_hbm, part_hbm, out_hbm,
               s_buf0, s_buf1, t_buf0, t_buf1, o_buf0, o_buf1, p_buf,
               sem0, sem1, osem0, osem1):
    base = _wid() * P
    sems = (sem0, sem1)
    osems = (osem0, osem1)
    s_bufs = (s_buf0, s_buf1)
    t_bufs = (t_buf0, t_buf1)
    o_bufs = (o_buf0, o_buf1)

    def start(ci):
        slot = ci % 2
        off = base + ci * C
        pltpu.async_copy(s_hbm.at[pl.ds(off, C)], s_bufs[slot], sems[slot])
        return pltpu.async_copy(
            t_hbm.at[pl.ds(off, C)], t_bufs[slot], sems[slot])

    h = start(0)

    pltpu.sync_copy(part_hbm, p_buf)

    def red(i, carry):
        a_all, a_1 = carry
        return (a_all + p_buf[pl.ds(i * 2 * L, L)],
                a_1 + p_buf[pl.ds(i * 2 * L + L, L)])

    z = jnp.zeros((L,), jnp.float32)
    acc_all, acc_1 = lax.fori_loop(0, NW, red, (z, z))
    r1 = _lane_allreduce(acc_1)
    r0 = _lane_allreduce(acc_all) - r1
    inv1 = 1.0 / r1
    inv0 = 1.0 / r0

    oh = (None, None)
    for ci in range(NCHUNK):
        h.wait()
        h.wait()
        if ci + 1 < NCHUNK:
            h = start(ci + 1)
        slot = ci % 2
        if oh[slot] is not None:
            oh[slot].wait()
        sb, tb, ob = s_bufs[slot], t_bufs[slot], o_bufs[slot]

        @plsc.parallel_loop(0, NV, step=U)
        def _(i):
            for j in range(U):
                v = jnp.exp(sb[pl.ds((i + j) * L, L)])
                t = tb[pl.ds((i + j) * L, L)]
                ob[pl.ds((i + j) * L, L)] = v * jnp.where(t == 1, inv1, inv0)

        off = base + ci * C
        new_oh = pltpu.async_copy(
            ob, out_hbm.at[pl.ds(off, C)], osems[slot])
        oh = (new_oh, oh[1]) if slot == 0 else (oh[0], new_oh)

    for hh in oh:
        if hh is not None:
            hh.wait()


def kernel(s, T):
    mesh = plsc.VectorSubcoreMesh(core_axis_name="c", subcore_axis_name="s")
    part = pl.kernel(
        _sum_body,
        mesh=mesh,
        out_type=jax.ShapeDtypeStruct((NW * 2 * L,), jnp.float32),
        scratch_types=[
            pltpu.VMEM((C,), jnp.float32),
            pltpu.VMEM((C,), jnp.float32),
            pltpu.VMEM((C,), jnp.int32),
            pltpu.VMEM((C,), jnp.int32),
            pltpu.VMEM((2 * L,), jnp.float32),
            pltpu.SemaphoreType.DMA,
            pltpu.SemaphoreType.DMA,
        ],
    )(s, T)
    out = pl.kernel(
        _norm_body,
        mesh=mesh,
        out_type=jax.ShapeDtypeStruct((N,), jnp.float32),
        scratch_types=[
            pltpu.VMEM((C,), jnp.float32),
            pltpu.VMEM((C,), jnp.float32),
            pltpu.VMEM((C,), jnp.int32),
            pltpu.VMEM((C,), jnp.int32),
            pltpu.VMEM((C,), jnp.float32),
            pltpu.VMEM((C,), jnp.float32),
            pltpu.VMEM((NW * 2 * L,), jnp.float32),
            pltpu.SemaphoreType.DMA,
            pltpu.SemaphoreType.DMA,
            pltpu.SemaphoreType.DMA,
            pltpu.SemaphoreType.DMA,
        ],
    )(s, T, part)
    return out
